# Initial kernel scaffold; baseline (speedup 1.0000x reference)
import jax
import jax.numpy as jnp
from jax.experimental import pallas as pl


def kernel(seq1, seq2, emb_table, W, b):
    raise NotImplementedError("write your pallas kernel here")


# speedup vs baseline: 4.4818x; 4.4818x over previous
import functools

import jax
import jax.numpy as jnp
from jax import lax
from jax.experimental import pallas as pl
from jax.experimental.pallas import tpu as pltpu
from jax.experimental.pallas import tpu_sc as plsc


def _project_body(w_ref, x_ref, o_ref):
    o_ref[...] = lax.dot_general(
        w_ref[...], x_ref[...],
        dimension_numbers=(((1,), (1,)), ((), ())),
        preferred_element_type=jnp.float32,
    )


def _project(emb, w2):
    V, D = emb.shape
    BLK = 32768
    grid = (pl.cdiv(V, BLK),)
    return pl.pallas_call(
        _project_body,
        grid=grid,
        in_specs=[
            pl.BlockSpec((2, D), lambda i: (0, 0)),
            pl.BlockSpec((BLK, D), lambda i: (i, 0)),
        ],
        out_specs=pl.BlockSpec((2, BLK), lambda i: (0, i)),
        out_shape=jax.ShapeDtypeStruct((2, V), jnp.float32),
    )(w2, emb)


def _make_sc(NC, NW, B, H):
    R = B // NW
    C = R // 16
    mesh = plsc.VectorSubcoreMesh(core_axis_name="c", subcore_axis_name="s")

    @functools.partial(
        pl.kernel,
        mesh=mesh,
        out_type=jax.ShapeDtypeStruct((B,), jnp.float32),
        scratch_types=[
            pltpu.VMEM((H * R,), jnp.int32),
            pltpu.VMEM((H * R,), jnp.float32),
            pltpu.VMEM((H * R,), jnp.int32),
            pltpu.VMEM((H * R,), jnp.float32),
            pltpu.VMEM((16,), jnp.float32),
            pltpu.VMEM((R,), jnp.float32),
            pltpu.SemaphoreType.DMA,
            pltpu.SemaphoreType.DMA,
        ],
    )
    def sc_kernel(t_hbm, idx1_hbm, idx2_hbm, b_hbm, out_hbm,
                  idx1_v, vals1_v, idx2_v, vals2_v, b_v, out_v, sem1, sem2):
        wid = lax.axis_index("s") * NC + lax.axis_index("c")
        base = wid * R
        pltpu.sync_copy(idx1_hbm.at[wid], idx1_v)
        cp1 = pltpu.async_copy(t_hbm.at[idx1_v], vals1_v, sem1)
        pltpu.sync_copy(idx2_hbm.at[wid], idx2_v)
        cp2 = pltpu.async_copy(t_hbm.at[idx2_v], vals2_v, sem2)
        pltpu.sync_copy(b_hbm, b_v)

        zeros = tuple(jnp.zeros((16,), jnp.float32) for _ in range(C))

        def accum(vals_v):
            def body(j, accs):
                return tuple(accs[c] + vals_v[pl.ds(j * R + c * 16, 16)]
                             for c in range(C))
            return body

        cp1.wait()
        acc = lax.fori_loop(0, H, accum(vals1_v), zeros)
        cp2.wait()
        acc = lax.fori_loop(0, H, accum(vals2_v), acc)

        bx = b_v[...]
        inv_h = jnp.float32(1.0 / H)
        for c in range(C):
            x = acc[c] * inv_h + bx
            out_v[pl.ds(c * 16, 16)] = 1.0 / (1.0 + jnp.exp(-x))
        pltpu.sync_copy(out_v, out_hbm.at[pl.ds(base, R)])

    return sc_kernel


def kernel(seq1, seq2, emb_table, W, b):
    V, D = emb_table.shape
    B, H = seq1.shape
    NC, NW = 2, 32

    w2 = W.reshape(2, D)
    t_flat = _project(emb_table, w2).reshape(2 * V)

    R = B // NW
    idx1 = seq1.astype(jnp.int32).T.reshape(H, NW, R).transpose(1, 0, 2).reshape(NW, H * R)
    idx2 = seq2.astype(jnp.int32).T.reshape(H, NW, R).transpose(1, 0, 2).reshape(NW, H * R) + V
    b16 = jnp.broadcast_to(b.astype(jnp.float32), (16,))

    out = _make_sc(NC, NW, B, H)(t_flat, idx1, idx2, b16)
    return out.reshape(B, 1)


# speedup vs baseline: 4.5337x; 1.0116x over previous
import functools

import jax
import jax.numpy as jnp
from jax import lax
from jax.experimental import pallas as pl
from jax.experimental.pallas import tpu as pltpu
from jax.experimental.pallas import tpu_sc as plsc


def _project_body(w_ref, x_ref, o1_ref, o2_ref):
    o = lax.dot_general(
        w_ref[...], x_ref[...],
        dimension_numbers=(((1,), (1,)), ((), ())),
        preferred_element_type=jnp.float32,
    )
    o1_ref[...] = o[0]
    o2_ref[...] = o[1]


def _project(emb, w2):
    V, D = emb.shape
    BLK = 32768
    grid = (pl.cdiv(V, BLK),)
    return pl.pallas_call(
        _project_body,
        grid=grid,
        in_specs=[
            pl.BlockSpec((2, D), lambda i: (0, 0)),
            pl.BlockSpec((BLK, D), lambda i: (i, 0)),
        ],
        out_specs=[
            pl.BlockSpec((BLK,), lambda i: (i,)),
            pl.BlockSpec((BLK,), lambda i: (i,)),
        ],
        out_shape=[
            jax.ShapeDtypeStruct((V,), jnp.float32),
            jax.ShapeDtypeStruct((V,), jnp.float32),
        ],
    )(w2, emb)


def _make_sc(NC, NW, B, H):
    R = B // NW
    C = R // 16
    mesh = plsc.VectorSubcoreMesh(core_axis_name="c", subcore_axis_name="s")

    @functools.partial(
        pl.kernel,
        mesh=mesh,
        out_type=jax.ShapeDtypeStruct((B,), jnp.float32),
        scratch_types=[
            pltpu.VMEM((H * R,), jnp.int32),
            pltpu.VMEM((H * R,), jnp.float32),
            pltpu.VMEM((H * R,), jnp.int32),
            pltpu.VMEM((H * R,), jnp.float32),
            pltpu.VMEM((16,), jnp.float32),
            pltpu.VMEM((R,), jnp.float32),
            pltpu.SemaphoreType.DMA,
            pltpu.SemaphoreType.DMA,
        ],
    )
    def sc_kernel(t1_hbm, t2_hbm, idx1_hbm, idx2_hbm, b_hbm, out_hbm,
                  idx1_v, vals1_v, idx2_v, vals2_v, b_v, out_v, sem1, sem2):
        wid = lax.axis_index("s") * NC + lax.axis_index("c")
        base = wid * R
        pltpu.sync_copy(idx1_hbm.at[wid], idx1_v)
        cp1 = pltpu.async_copy(t1_hbm.at[idx1_v], vals1_v, sem1)
        pltpu.sync_copy(idx2_hbm.at[wid], idx2_v)
        cp2 = pltpu.async_copy(t2_hbm.at[idx2_v], vals2_v, sem2)
        pltpu.sync_copy(b_hbm, b_v)

        zeros = tuple(jnp.zeros((16,), jnp.float32) for _ in range(C))

        def accum(vals_v):
            def body(j, accs):
                return tuple(
                    accs[c] + vals_v[pl.ds(j * R + c * 16, 16)]
                    for c in range(C)
                )
            return body

        cp1.wait()
        acc = lax.fori_loop(0, H, accum(vals1_v), zeros)
        cp2.wait()
        acc = lax.fori_loop(0, H, accum(vals2_v), acc)

        bx = b_v[...]
        inv_h = jnp.float32(1.0 / H)
        for c in range(C):
            x = acc[c] * inv_h + bx
            out_v[pl.ds(c * 16, 16)] = 1.0 / (1.0 + jnp.exp(-x))
        pltpu.sync_copy(out_v, out_hbm.at[pl.ds(base, R)])

    return sc_kernel


def kernel(seq1, seq2, emb_table, W, b):
    V, D = emb_table.shape
    B, H = seq1.shape
    NC, NW = 2, 32

    w2 = W.reshape(2, D)
    t1, t2 = _project(emb_table, w2)

    R = B // NW
    idx1 = seq1.astype(jnp.int32).T.reshape(H, NW, R).transpose(1, 0, 2).reshape(NW, H * R)
    idx2 = seq2.astype(jnp.int32).T.reshape(H, NW, R).transpose(1, 0, 2).reshape(NW, H * R)
    b16 = jnp.broadcast_to(b.astype(jnp.float32), (16,))

    out = _make_sc(NC, NW, B, H)(t1, t2, idx1, idx2, b16)
    return out.reshape(B, 1)


# speedup vs baseline: 4.5400x; 1.0014x over previous
import functools

import jax
import jax.numpy as jnp
from jax import lax
from jax.experimental import pallas as pl
from jax.experimental.pallas import tpu as pltpu
from jax.experimental.pallas import tpu_sc as plsc


def _project_body(w_ref, x_ref, o1_ref, o2_ref):
    o = lax.dot_general(
        w_ref[...], x_ref[...],
        dimension_numbers=(((1,), (1,)), ((), ())),
        preferred_element_type=jnp.float32,
    )
    o1_ref[...] = o[0]
    o2_ref[...] = o[1]


def _project(emb, w2):
    V, D = emb.shape
    BLK = 16384
    grid = (pl.cdiv(V, BLK),)
    return pl.pallas_call(
        _project_body,
        grid=grid,
        in_specs=[
            pl.BlockSpec((2, D), lambda i: (0, 0)),
            pl.BlockSpec((BLK, D), lambda i: (i, 0)),
        ],
        out_specs=[
            pl.BlockSpec((BLK,), lambda i: (i,)),
            pl.BlockSpec((BLK,), lambda i: (i,)),
        ],
        out_shape=[
            jax.ShapeDtypeStruct((V,), jnp.float32),
            jax.ShapeDtypeStruct((V,), jnp.float32),
        ],
    )(w2, emb)


def _make_sc(NC, NW, B, H):
    R = B // NW
    C = R // 16
    mesh = plsc.VectorSubcoreMesh(core_axis_name="c", subcore_axis_name="s")

    @functools.partial(
        pl.kernel,
        mesh=mesh,
        out_type=jax.ShapeDtypeStruct((B,), jnp.float32),
        scratch_types=[
            pltpu.VMEM((H * R,), jnp.int32),
            pltpu.VMEM((H * R,), jnp.float32),
            pltpu.VMEM((H * R,), jnp.int32),
            pltpu.VMEM((H * R,), jnp.float32),
            pltpu.VMEM((16,), jnp.float32),
            pltpu.VMEM((R,), jnp.float32),
            pltpu.SemaphoreType.DMA,
            pltpu.SemaphoreType.DMA,
        ],
    )
    def sc_kernel(t1_hbm, t2_hbm, idx1_hbm, idx2_hbm, b_hbm, out_hbm,
                  idx1_v, vals1_v, idx2_v, vals2_v, b_v, out_v, sem1, sem2):
        wid = lax.axis_index("s") * NC + lax.axis_index("c")
        base = wid * R
        pltpu.sync_copy(idx1_hbm.at[wid], idx1_v)
        cp1 = pltpu.async_copy(t1_hbm.at[idx1_v], vals1_v, sem1)
        pltpu.sync_copy(idx2_hbm.at[wid], idx2_v)
        cp2 = pltpu.async_copy(t2_hbm.at[idx2_v], vals2_v, sem2)
        pltpu.sync_copy(b_hbm, b_v)

        zeros = tuple(jnp.zeros((16,), jnp.float32) for _ in range(C))

        def accum(vals_v):
            def body(j, accs):
                return tuple(
                    accs[c] + vals_v[pl.ds(j * R + c * 16, 16)]
                    for c in range(C)
                )
            return body

        cp1.wait()
        acc = lax.fori_loop(0, H, accum(vals1_v), zeros)
        cp2.wait()
        acc = lax.fori_loop(0, H, accum(vals2_v), acc)

        bx = b_v[...]
        inv_h = jnp.float32(1.0 / H)
        for c in range(C):
            x = acc[c] * inv_h + bx
            out_v[pl.ds(c * 16, 16)] = 1.0 / (1.0 + jnp.exp(-x))
        pltpu.sync_copy(out_v, out_hbm.at[pl.ds(base, R)])

    return sc_kernel


def kernel(seq1, seq2, emb_table, W, b):
    V, D = emb_table.shape
    B, H = seq1.shape
    NC, NW = 2, 32

    w2 = W.reshape(2, D)
    t1, t2 = _project(emb_table, w2)

    R = B // NW
    idx1 = seq1.astype(jnp.int32).T.reshape(H, NW, R).transpose(1, 0, 2).reshape(NW, H * R)
    idx2 = seq2.astype(jnp.int32).T.reshape(H, NW, R).transpose(1, 0, 2).reshape(NW, H * R)
    b16 = jnp.broadcast_to(b.astype(jnp.float32), (16,))

    out = _make_sc(NC, NW, B, H)(t1, t2, idx1, idx2, b16)
    return out.reshape(B, 1)
